# Initial kernel scaffold; baseline (speedup 1.0000x reference)
import jax
import jax.numpy as jnp
from jax.experimental import pallas as pl


def kernel(xyz1, xyz2, normal_rebuild, normal_gt):
    raise NotImplementedError("write your pallas kernel here")


# speedup vs baseline: 2.1446x; 2.1446x over previous
import functools

import jax
import jax.numpy as jnp
from jax.experimental import pallas as pl

_EPS = 1e-12


def _chamfer_body(x1_ref, x2t_ref, n1_ref, n2t_ref, xyz_ref, nrm_ref,
                  *, B, N, M, C):
    b = pl.program_id(0)
    x2t = x2t_ref[0]
    n2t = n2t_ref[0]

    x2sq = jnp.sum(x2t * x2t, axis=0, keepdims=True)

    u2t = n2t / jnp.maximum(
        jnp.sqrt(jnp.sum(n2t * n2t, axis=0, keepdims=True)), _EPS)
    s2 = jnp.sum(u2t * u2t, axis=0, keepdims=True)

    inf = jnp.float32(jnp.inf)

    def body(i, carry):
        colmin, colnd, sum_d1, sum_nd1 = carry
        x1c = x1_ref[0, pl.ds(i * C, C), :]
        n1c = n1_ref[0, pl.ds(i * C, C), :]
        x1sqc = jnp.sum(x1c * x1c, axis=1, keepdims=True)
        u1c = n1c / jnp.maximum(
            jnp.sqrt(jnp.sum(n1c * n1c, axis=1, keepdims=True)), _EPS)
        s1c = jnp.sum(u1c * u1c, axis=1, keepdims=True)

        g = jax.lax.dot_general(x1c, x2t, (((1,), (0,)), ((), ())),
                                preferred_element_type=jnp.float32,
                                precision=jax.lax.Precision.DEFAULT)
        d = x1sqc + x2sq - 2.0 * g

        h = jax.lax.dot_general(u1c, u2t, (((1,), (0,)), ((), ())),
                                preferred_element_type=jnp.float32,
                                precision=jax.lax.Precision.DEFAULT)
        nd = s1c + s2 - 2.0 * jnp.abs(h)

        rowmin = jnp.min(d, axis=1, keepdims=True)
        nd1sel = jnp.min(jnp.where(d == rowmin, nd, inf),
                         axis=1, keepdims=True)
        sum_d1 = sum_d1 + jnp.sum(rowmin)
        sum_nd1 = sum_nd1 + jnp.sum(nd1sel)

        colmin_c = jnp.min(d, axis=0, keepdims=True)
        colnd_c = jnp.min(jnp.where(d == colmin_c, nd, inf),
                          axis=0, keepdims=True)
        colnd = jnp.where(colmin_c < colmin, colnd_c, colnd)
        colmin = jnp.minimum(colmin_c, colmin)
        return colmin, colnd, sum_d1, sum_nd1

    zero = jnp.zeros((), jnp.float32)
    init = (jnp.full((1, M), inf, jnp.float32),
            jnp.full((1, M), inf, jnp.float32), zero, zero)
    colmin, colnd, sum_d1, sum_nd1 = jax.lax.fori_loop(0, N // C, body, init)

    sum_d2 = jnp.sum(colmin)
    sum_nd2 = jnp.sum(colnd)

    loss_xyz_part = sum_d1 / (B * N) + sum_d2 / (B * M)
    loss_nrm_part = sum_nd1 / (B * N) + sum_nd2 / (B * M)

    @pl.when(b == 0)
    def _():
        xyz_ref[...] = jnp.zeros((1, 1), jnp.float32)
        nrm_ref[...] = jnp.zeros((1, 1), jnp.float32)

    xyz_ref[...] += jnp.reshape(loss_xyz_part, (1, 1))
    nrm_ref[...] += jnp.reshape(loss_nrm_part, (1, 1))


def kernel(xyz1, xyz2, normal_rebuild, normal_gt):
    B, N, _ = xyz1.shape
    M = xyz2.shape[1]
    C = 256
    x2t = jnp.transpose(xyz2, (0, 2, 1))
    n2t = jnp.transpose(normal_gt, (0, 2, 1))
    out = pl.pallas_call(
        functools.partial(_chamfer_body, B=B, N=N, M=M, C=C),
        grid=(B,),
        in_specs=[
            pl.BlockSpec((1, N, 3), lambda b: (b, 0, 0)),
            pl.BlockSpec((1, 3, M), lambda b: (b, 0, 0)),
            pl.BlockSpec((1, N, 3), lambda b: (b, 0, 0)),
            pl.BlockSpec((1, 3, M), lambda b: (b, 0, 0)),
        ],
        out_specs=[pl.BlockSpec((1, 1), lambda b: (0, 0)),
                   pl.BlockSpec((1, 1), lambda b: (0, 0))],
        out_shape=[jax.ShapeDtypeStruct((1, 1), jnp.float32),
                   jax.ShapeDtypeStruct((1, 1), jnp.float32)],
    )(xyz1, x2t, normal_rebuild, n2t)
    return (out[0][0, 0], out[1][0, 0])


# speedup vs baseline: 2.6760x; 1.2478x over previous
import functools

import jax
import jax.numpy as jnp
from jax.experimental import pallas as pl

_EPS = 1e-12


def _chamfer_body(x1_ref, x2t_ref, n1_ref, n2t_ref, xyz_ref, nrm_ref,
                  *, B, N, M, C):
    b = pl.program_id(0)
    x2t = x2t_ref[0]
    n2t = n2t_ref[0]

    x2sq = jnp.sum(x2t * x2t, axis=0, keepdims=True)

    u2t = n2t / jnp.maximum(
        jnp.sqrt(jnp.sum(n2t * n2t, axis=0, keepdims=True)), _EPS)

    inf = jnp.float32(jnp.inf)

    def body(i, carry):
        colmin, colpick, sum_d1, sum_pick1 = carry
        x1c = x1_ref[0, pl.ds(i * C, C), :]
        n1c = n1_ref[0, pl.ds(i * C, C), :]
        x1sqc = jnp.sum(x1c * x1c, axis=1, keepdims=True)
        u1c2 = n1c / jnp.maximum(
            0.5 * jnp.sqrt(jnp.sum(n1c * n1c, axis=1, keepdims=True)),
            0.5 * _EPS)

        g2 = jax.lax.dot_general(2.0 * x1c, x2t, (((1,), (0,)), ((), ())),
                                 preferred_element_type=jnp.float32)
        d = (x1sqc + x2sq) - g2

        h2 = jax.lax.dot_general(u1c2, u2t, (((1,), (0,)), ((), ())),
                                 preferred_element_type=jnp.float32)
        a = jnp.abs(h2)

        rowmin = jnp.min(d, axis=1, keepdims=True)
        pick1 = jnp.max(jnp.where(d == rowmin, a, -inf),
                        axis=1, keepdims=True)
        sum_d1 = sum_d1 + jnp.sum(rowmin)
        sum_pick1 = sum_pick1 + jnp.sum(pick1)

        colmin_c = jnp.min(d, axis=0, keepdims=True)
        colpick_c = jnp.max(jnp.where(d == colmin_c, a, -inf),
                            axis=0, keepdims=True)
        colpick = jnp.where(colmin_c < colmin, colpick_c, colpick)
        colmin = jnp.minimum(colmin_c, colmin)
        return colmin, colpick, sum_d1, sum_pick1

    zero = jnp.zeros((), jnp.float32)
    init = (jnp.full((1, M), inf, jnp.float32),
            jnp.full((1, M), -inf, jnp.float32), zero, zero)
    colmin, colpick, sum_d1, sum_pick1 = jax.lax.fori_loop(
        0, N // C, body, init)

    sum_d2 = jnp.sum(colmin)
    sum_nd1 = 2.0 * N - sum_pick1
    sum_nd2 = 2.0 * M - jnp.sum(colpick)

    loss_xyz_part = sum_d1 / (B * N) + sum_d2 / (B * M)
    loss_nrm_part = sum_nd1 / (B * N) + sum_nd2 / (B * M)

    @pl.when(b == 0)
    def _():
        xyz_ref[...] = jnp.zeros((1, 1), jnp.float32)
        nrm_ref[...] = jnp.zeros((1, 1), jnp.float32)

    xyz_ref[...] += jnp.reshape(loss_xyz_part, (1, 1))
    nrm_ref[...] += jnp.reshape(loss_nrm_part, (1, 1))


def kernel(xyz1, xyz2, normal_rebuild, normal_gt):
    B, N, _ = xyz1.shape
    M = xyz2.shape[1]
    C = 256
    x2t = jnp.transpose(xyz2, (0, 2, 1))
    n2t = jnp.transpose(normal_gt, (0, 2, 1))
    out = pl.pallas_call(
        functools.partial(_chamfer_body, B=B, N=N, M=M, C=C),
        grid=(B,),
        in_specs=[
            pl.BlockSpec((1, N, 3), lambda b: (b, 0, 0)),
            pl.BlockSpec((1, 3, M), lambda b: (b, 0, 0)),
            pl.BlockSpec((1, N, 3), lambda b: (b, 0, 0)),
            pl.BlockSpec((1, 3, M), lambda b: (b, 0, 0)),
        ],
        out_specs=[pl.BlockSpec((1, 1), lambda b: (0, 0)),
                   pl.BlockSpec((1, 1), lambda b: (0, 0))],
        out_shape=[jax.ShapeDtypeStruct((1, 1), jnp.float32),
                   jax.ShapeDtypeStruct((1, 1), jnp.float32)],
    )(xyz1, x2t, normal_rebuild, n2t)
    return (out[0][0, 0], out[1][0, 0])
